# Initial kernel scaffold; baseline (speedup 1.0000x reference)
import functools

import jax
import jax.numpy as jnp
from jax import lax
from jax.experimental import pallas as pl
from jax.experimental.pallas import tpu as pltpu
from jax.experimental.pallas import tpu_sc as plsc

N = 10000
H = 128
E = 320000
PW = 16
TW = H + PW
W = 128
NC = 2
NS = 16
ROWS_PER_TILE = N // NS

EDGE_BLK = 2000


def _tables_body(h_ref, posp_ref, w1a_ref, w1b_ref, be1_ref, wd_ref,
                 d_ref, s_ref):
    h = h_ref[...]
    posp = posp_ref[...]
    p2 = jnp.sum(posp * posp, axis=1, keepdims=True)
    wd = wd_ref[...]
    pterm = p2 * wd
    d_ref[:, :H] = (jnp.dot(h, w1a_ref[...], preferred_element_type=jnp.float32)
                    + be1_ref[...] + pterm)
    d_ref[:, H:] = posp
    s_ref[:, :H] = (jnp.dot(h, w1b_ref[...], preferred_element_type=jnp.float32)
                    + pterm)
    s_ref[:, H:] = -2.0 * posp


def _build_tables(h2, posp, w1a, w1b, be1, wd):
    return pl.pallas_call(
        _tables_body,
        out_shape=(jax.ShapeDtypeStruct((N, TW), jnp.float32),
                   jax.ShapeDtypeStruct((N, TW), jnp.float32)),
    )(h2, posp, w1a, w1b, be1, wd)


def _sc_gather(d_tab, s_tab, dsti, srci):
    mesh = plsc.VectorSubcoreMesh(core_axis_name="c", subcore_axis_name="s")

    @functools.partial(
        pl.kernel,
        out_type=(jax.ShapeDtypeStruct((E, TW), jnp.float32),
                  jax.ShapeDtypeStruct((E, TW), jnp.float32)),
        mesh=mesh,
    )
    def k(d_hbm, s_hbm, di_hbm, si_hbm, dg_hbm, sg_hbm):
        def body(di_v, si_v, dg_v, sg_v):
            pltpu.sync_copy(d_hbm.at[di_v.at[0]], dg_v)
            pltpu.sync_copy(s_hbm.at[si_v.at[0]], sg_v)

        pltpu.emit_pipeline(
            body,
            grid=(E // W,),
            in_specs=[pl.BlockSpec((1, W), lambda i: (0, i)),
                      pl.BlockSpec((1, W), lambda i: (0, i))],
            out_specs=[pl.BlockSpec((W, TW), lambda i: (i, 0)),
                       pl.BlockSpec((W, TW), lambda i: (i, 0))],
            core_axis_name=("c", "s"),
            dimension_semantics=(pltpu.PARALLEL,),
        )(di_hbm, si_hbm, dg_hbm, sg_hbm)

    return k(d_tab, s_tab, dsti, srci)


def _edge_mlp_body(dg_ref, sg_ref, we2_ref, wd_ref, be2_ref, out_ref):
    dg = dg_ref[...]
    sg = sg_ref[...]
    dot = jnp.sum(dg[:, H:] * sg[:, H:], axis=1, keepdims=True)
    pre = dg[:, :H] + sg[:, :H] + dot * wd_ref[...]
    g1 = jax.nn.gelu(pre, approximate=False)
    g2 = (jnp.dot(g1, we2_ref[...], preferred_element_type=jnp.float32)
          + be2_ref[...])
    out_ref[...] = jax.nn.gelu(g2, approximate=False)


def _edge_mlp(dg, sg, we2, wd, be2):
    nblk = E // EDGE_BLK
    return pl.pallas_call(
        _edge_mlp_body,
        grid=(nblk,),
        in_specs=[
            pl.BlockSpec((EDGE_BLK, TW), lambda i: (i, 0)),
            pl.BlockSpec((EDGE_BLK, TW), lambda i: (i, 0)),
            pl.BlockSpec((H, H), lambda i: (0, 0)),
            pl.BlockSpec((1, H), lambda i: (0, 0)),
            pl.BlockSpec((1, H), lambda i: (0, 0)),
        ],
        out_specs=pl.BlockSpec((EDGE_BLK, H), lambda i: (i, 0)),
        out_shape=jax.ShapeDtypeStruct((E, H), jnp.float32),
    )(dg, sg, we2, wd, be2)


def _sc_scatter(e2, dsti, zrows):
    mesh = plsc.VectorSubcoreMesh(core_axis_name="c", subcore_axis_name="s")

    @functools.partial(
        pl.kernel,
        out_type=jax.ShapeDtypeStruct((NC, N, H), jnp.float32),
        mesh=mesh,
        scratch_types=[pltpu.VMEM_SHARED((N, H), jnp.float32)],
    )
    def k(e2_hbm, di_hbm, z_hbm, out_hbm, acc_sh):
        sid = lax.axis_index("s")
        cid = lax.axis_index("c")
        rs = sid * ROWS_PER_TILE
        pltpu.sync_copy(z_hbm.at[pl.ds(rs, ROWS_PER_TILE)],
                        acc_sh.at[pl.ds(rs, ROWS_PER_TILE)])
        plsc.subcore_barrier()

        def body(e2_v, di_v):
            pltpu.sync_copy(e2_v, acc_sh.at[di_v.at[0]], add=True)

        pltpu.emit_pipeline(
            body,
            grid=(E // W,),
            in_specs=[pl.BlockSpec((W, H), lambda i: (i, 0)),
                      pl.BlockSpec((1, W), lambda i: (0, i))],
            out_specs=[],
            core_axis_name=("c", "s"),
            dimension_semantics=(pltpu.PARALLEL,),
        )(e2_hbm, di_hbm)
        plsc.subcore_barrier()
        pltpu.sync_copy(acc_sh.at[pl.ds(rs, ROWS_PER_TILE)],
                        out_hbm.at[cid, pl.ds(rs, ROWS_PER_TILE)])

    return k(e2, dsti, zrows)


def _node_mlp_body(h_ref, p0_ref, p1_ref, wv1h_ref, wv1a_ref, bv1_ref,
                   wv2_ref, bv2_ref, out_ref):
    h = h_ref[...]
    agg = p0_ref[...] + p1_ref[...]
    v = (jnp.dot(h, wv1h_ref[...], preferred_element_type=jnp.float32)
         + jnp.dot(agg, wv1a_ref[...], preferred_element_type=jnp.float32)
         + bv1_ref[...])
    v = jax.nn.gelu(v, approximate=False)
    v = (jnp.dot(v, wv2_ref[...], preferred_element_type=jnp.float32)
         + bv2_ref[...])
    v = jax.nn.gelu(v, approximate=False)
    out_ref[...] = h + v


def _node_mlp(h2, p0, p1, wv1h, wv1a, bv1, wv2, bv2):
    return pl.pallas_call(
        _node_mlp_body,
        out_shape=jax.ShapeDtypeStruct((N, H), jnp.float32),
    )(h2, p0, p1, wv1h, wv1a, bv1, wv2, bv2)


def kernel(h, pos, edge_index, W_e1, b_e1, W_e2, b_e2, W_v1, b_v1, W_v2, b_v2):
    h2 = h[0]
    posp = jnp.pad(pos[0], ((0, 0), (0, PW - 3)))
    src = edge_index[0].reshape(1, E)
    dst = edge_index[1].reshape(1, E)
    w1a = W_e1[:H]
    w1b = W_e1[H:2 * H]
    wd = W_e1[2 * H].reshape(1, H)

    d_tab, s_tab = _build_tables(h2, posp, w1a, w1b, b_e1.reshape(1, H), wd)
    dg, sg = _sc_gather(d_tab, s_tab, dst, src)
    e2 = _edge_mlp(dg, sg, W_e2, wd, b_e2.reshape(1, H))
    parts = _sc_scatter(e2, dst, jnp.zeros((N, H), jnp.float32))
    out = _node_mlp(h2, parts[0], parts[1], W_v1[:H], W_v1[H:],
                    b_v1.reshape(1, H), W_v2, b_v2.reshape(1, H))
    return out.reshape(1, N, H)


# speedup vs baseline: 3.7457x; 3.7457x over previous
import functools

import jax
import jax.numpy as jnp
from jax import lax
from jax.experimental import pallas as pl
from jax.experimental.pallas import tpu as pltpu
from jax.experimental.pallas import tpu_sc as plsc

N = 10000
H = 128
E = 320000
PW = 16
W = 80
NC = 2
NS = 16
ROWS_PER_TILE = N // NS

EDGE_BLK = 2000

_INV_SQRT2 = 0.7071067811865476


def _gelu(x):
    return 0.5 * x * (1.0 + lax.erf(x * _INV_SQRT2))


def _tables_body(h_ref, posp_ref, w1a_ref, w1b_ref, be1_ref, wd_ref,
                 d_ref, s_ref, pd_ref, ps_ref):
    h = h_ref[...]
    posp = posp_ref[...]
    p2 = jnp.sum(posp * posp, axis=1, keepdims=True)
    wd = wd_ref[...]
    pterm = p2 * wd
    d_ref[...] = (jnp.dot(h, w1a_ref[...], preferred_element_type=jnp.float32)
                  + be1_ref[...] + pterm)
    s_ref[...] = (jnp.dot(h, w1b_ref[...], preferred_element_type=jnp.float32)
                  + pterm)
    pd_ref[...] = posp
    ps_ref[...] = -2.0 * posp


def _build_tables(h2, posp, w1a, w1b, be1, wd):
    return pl.pallas_call(
        _tables_body,
        out_shape=(jax.ShapeDtypeStruct((N, H), jnp.float32),
                   jax.ShapeDtypeStruct((N, H), jnp.float32),
                   jax.ShapeDtypeStruct((N, PW), jnp.float32),
                   jax.ShapeDtypeStruct((N, PW), jnp.float32)),
    )(h2, posp, w1a, w1b, be1, wd)


def _sc_gather(d_tab, s_tab, pd_tab, ps_tab, dsti, srci):
    mesh = plsc.VectorSubcoreMesh(core_axis_name="c", subcore_axis_name="s")

    @functools.partial(
        pl.kernel,
        out_type=(jax.ShapeDtypeStruct((E, H), jnp.float32),
                  jax.ShapeDtypeStruct((E, H), jnp.float32),
                  jax.ShapeDtypeStruct((E, PW), jnp.float32),
                  jax.ShapeDtypeStruct((E, PW), jnp.float32)),
        mesh=mesh,
        compiler_params=pltpu.CompilerParams(use_tc_tiling_on_sc=False),
    )
    def k(d_hbm, s_hbm, pd_hbm, ps_hbm, di_hbm, si_hbm,
          dg_hbm, sg_hbm, pdg_hbm, psg_hbm):
        def body(di_v, si_v, dg_v, sg_v, pdg_v, psg_v):
            pltpu.sync_copy(d_hbm.at[di_v.at[0]], dg_v)
            pltpu.sync_copy(s_hbm.at[si_v.at[0]], sg_v)
            pltpu.sync_copy(pd_hbm.at[di_v.at[0]], pdg_v)
            pltpu.sync_copy(ps_hbm.at[si_v.at[0]], psg_v)

        pltpu.emit_pipeline(
            body,
            grid=(E // W,),
            in_specs=[pl.BlockSpec((1, W), lambda i: (0, i)),
                      pl.BlockSpec((1, W), lambda i: (0, i))],
            out_specs=[pl.BlockSpec((W, H), lambda i: (i, 0)),
                       pl.BlockSpec((W, H), lambda i: (i, 0)),
                       pl.BlockSpec((W, PW), lambda i: (i, 0)),
                       pl.BlockSpec((W, PW), lambda i: (i, 0))],
            core_axis_name=("c", "s"),
            dimension_semantics=(pltpu.PARALLEL,),
        )(di_hbm, si_hbm, dg_hbm, sg_hbm, pdg_hbm, psg_hbm)

    return k(d_tab, s_tab, pd_tab, ps_tab, dsti, srci)


def _edge_mlp_body(dg_ref, sg_ref, pdg_ref, psg_ref, we2_ref, wd_ref,
                   be2_ref, out_ref):
    dg = dg_ref[...]
    sg = sg_ref[...]
    dot = jnp.sum(pdg_ref[...] * psg_ref[...], axis=1, keepdims=True)
    pre = dg + sg + dot * wd_ref[...]
    g1 = _gelu(pre)
    g2 = (jnp.dot(g1, we2_ref[...], preferred_element_type=jnp.float32)
          + be2_ref[...])
    out_ref[...] = _gelu(g2)


def _edge_mlp(dg, sg, pdg, psg, we2, wd, be2):
    nblk = E // EDGE_BLK
    return pl.pallas_call(
        _edge_mlp_body,
        grid=(nblk,),
        in_specs=[
            pl.BlockSpec((EDGE_BLK, H), lambda i: (i, 0)),
            pl.BlockSpec((EDGE_BLK, H), lambda i: (i, 0)),
            pl.BlockSpec((EDGE_BLK, PW), lambda i: (i, 0)),
            pl.BlockSpec((EDGE_BLK, PW), lambda i: (i, 0)),
            pl.BlockSpec((H, H), lambda i: (0, 0)),
            pl.BlockSpec((1, H), lambda i: (0, 0)),
            pl.BlockSpec((1, H), lambda i: (0, 0)),
        ],
        out_specs=pl.BlockSpec((EDGE_BLK, H), lambda i: (i, 0)),
        out_shape=jax.ShapeDtypeStruct((E, H), jnp.float32),
    )(dg, sg, pdg, psg, we2, wd, be2)


def _sc_scatter(e2, dsti, zrows):
    mesh = plsc.VectorSubcoreMesh(core_axis_name="c", subcore_axis_name="s")

    @functools.partial(
        pl.kernel,
        out_type=jax.ShapeDtypeStruct((NC, N, H), jnp.float32),
        mesh=mesh,
        scratch_types=[pltpu.VMEM_SHARED((N, H), jnp.float32)],
        compiler_params=pltpu.CompilerParams(use_tc_tiling_on_sc=False),
    )
    def k(e2_hbm, di_hbm, z_hbm, out_hbm, acc_sh):
        sid = lax.axis_index("s")
        cid = lax.axis_index("c")
        rs = sid * ROWS_PER_TILE
        pltpu.sync_copy(z_hbm.at[pl.ds(rs, ROWS_PER_TILE)],
                        acc_sh.at[pl.ds(rs, ROWS_PER_TILE)])
        plsc.subcore_barrier()

        def body(e2_v, di_v):
            pltpu.sync_copy(e2_v, acc_sh.at[di_v.at[0]], add=True)

        pltpu.emit_pipeline(
            body,
            grid=(E // W,),
            in_specs=[pl.BlockSpec((W, H), lambda i: (i, 0)),
                      pl.BlockSpec((1, W), lambda i: (0, i))],
            out_specs=[],
            core_axis_name=("c", "s"),
            dimension_semantics=(pltpu.PARALLEL,),
        )(e2_hbm, di_hbm)
        plsc.subcore_barrier()
        pltpu.sync_copy(acc_sh.at[pl.ds(rs, ROWS_PER_TILE)],
                        out_hbm.at[cid, pl.ds(rs, ROWS_PER_TILE)])

    return k(e2, dsti, zrows)


def _node_mlp_body(h_ref, p0_ref, p1_ref, wv1h_ref, wv1a_ref, bv1_ref,
                   wv2_ref, bv2_ref, out_ref):
    h = h_ref[...]
    agg = p0_ref[...] + p1_ref[...]
    v = (jnp.dot(h, wv1h_ref[...], preferred_element_type=jnp.float32)
         + jnp.dot(agg, wv1a_ref[...], preferred_element_type=jnp.float32)
         + bv1_ref[...])
    v = _gelu(v)
    v = (jnp.dot(v, wv2_ref[...], preferred_element_type=jnp.float32)
         + bv2_ref[...])
    v = _gelu(v)
    out_ref[...] = h + v


def _node_mlp(h2, p0, p1, wv1h, wv1a, bv1, wv2, bv2):
    return pl.pallas_call(
        _node_mlp_body,
        out_shape=jax.ShapeDtypeStruct((N, H), jnp.float32),
    )(h2, p0, p1, wv1h, wv1a, bv1, wv2, bv2)


def kernel(h, pos, edge_index, W_e1, b_e1, W_e2, b_e2, W_v1, b_v1, W_v2, b_v2):
    h2 = h[0]
    posp = jnp.pad(pos[0], ((0, 0), (0, PW - 3)))
    src = edge_index[0].reshape(1, E)
    dst = edge_index[1].reshape(1, E)
    w1a = W_e1[:H]
    w1b = W_e1[H:2 * H]
    wd = W_e1[2 * H].reshape(1, H)

    d_tab, s_tab, pd_tab, ps_tab = _build_tables(
        h2, posp, w1a, w1b, b_e1.reshape(1, H), wd)
    dg, sg, pdg, psg = _sc_gather(d_tab, s_tab, pd_tab, ps_tab, dst, src)
    e2 = _edge_mlp(dg, sg, pdg, psg, W_e2, wd, b_e2.reshape(1, H))
    parts = _sc_scatter(e2, dst, jnp.zeros((N, H), jnp.float32))
    out = _node_mlp(h2, parts[0], parts[1], W_v1[:H], W_v1[H:],
                    b_v1.reshape(1, H), W_v2, b_v2.reshape(1, H))
    return out.reshape(1, N, H)
